# Initial kernel scaffold; baseline (speedup 1.0000x reference)
import jax
import jax.numpy as jnp
from jax.experimental import pallas as pl


def kernel(x, edge_index, W1, b1, W2, b2):
    raise NotImplementedError("write your pallas kernel here")


# speedup vs baseline: 10.2680x; 10.2680x over previous
import functools

import jax
import jax.numpy as jnp
from jax import lax
from jax.experimental import pallas as pl
from jax.experimental.pallas import tpu as pltpu
from jax.experimental.pallas import tpu_sc as plsc

N = 10000
E = 320000
IN_C = 128
HID = 16
OUT_C = 64
K = 10
ALPHA = 0.1

NC = 2
NS = 16
W = NC * NS
C = 128
CH = 80
EPW = C * CH
E_PAD = W * EPW
NP = 10112
RPT = NP // NS

_mesh = plsc.VectorSubcoreMesh(
    core_axis_name="c", subcore_axis_name="s", num_cores=NC, num_subcores=NS
)


def _mlp_body(x_ref, w1_ref, b1_ref, w2_ref, b2_ref, h_ref):
    h1 = jnp.dot(x_ref[...], w1_ref[...], preferred_element_type=jnp.float32,
                 precision=lax.Precision.HIGHEST)
    h1 = jnp.maximum(h1 + b1_ref[...], 0.0)
    h2 = jnp.dot(h1, w2_ref[...], preferred_element_type=jnp.float32,
                 precision=lax.Precision.HIGHEST)
    h_ref[...] = h2 + b2_ref[...]


def _prep_body(deg16_ref, h_ref, dinv_ref, dinv2_ref, p0_ref):
    degs = deg16_ref[0] + deg16_ref[1]
    deg = jnp.sum(degs, axis=1, keepdims=True) * (1.0 / 16.0) + 1.0
    rows = lax.broadcasted_iota(jnp.int32, (NP, 1), 0)
    dinv = jnp.where(rows < N, lax.rsqrt(deg), 0.0)
    dinv_ref[...] = dinv
    dinv2_ref[...] = dinv * dinv
    p0_ref[...] = dinv * h_ref[...]


def _blend_body(agg_ref, out_ref, h_ref, dinv_ref, dinv2_ref,
                out_next_ref, p_next_ref):
    agg = agg_ref[0] + agg_ref[1]
    o = (1.0 - ALPHA) * (dinv_ref[...] * agg + dinv2_ref[...] * out_ref[...])
    o = o + ALPHA * h_ref[...]
    out_next_ref[...] = o
    p_next_ref[...] = dinv_ref[...] * o


def _lsm_body(o_ref, y_ref):
    o = o_ref[...]
    m = jnp.max(o, axis=1, keepdims=True)
    y = o - m
    y_ref[...] = y - jnp.log(jnp.sum(jnp.exp(y), axis=1, keepdims=True))


def _deg_body(dst_hbm, ones_hbm, zeros_hbm, out_hbm, dst_vm, ones_vm, deg_sh):
    c = lax.axis_index("c")
    s = lax.axis_index("s")
    w = c * NS + s
    pltpu.sync_copy(zeros_hbm.at[pl.ds(s * RPT, RPT)],
                    deg_sh.at[pl.ds(s * RPT, RPT)])
    pltpu.sync_copy(ones_hbm, ones_vm)
    pltpu.sync_copy(dst_hbm.at[w], dst_vm)
    plsc.subcore_barrier()

    @pl.loop(0, CH)
    def _chunks(j):
        pltpu.sync_copy(ones_vm, deg_sh.at[dst_vm.at[j]], add=True)

    plsc.subcore_barrier()
    pltpu.sync_copy(deg_sh.at[pl.ds(s * RPT, RPT)],
                    out_hbm.at[c, pl.ds(s * RPT, RPT)])


def _spmm_body(p_hbm, src_hbm, dst_hbm, zeros_hbm, out_hbm,
               src_vm, dst_vm, buf_a, buf_b, agg_sh, sem_a, sem_b):
    c = lax.axis_index("c")
    s = lax.axis_index("s")
    w = c * NS + s
    pltpu.sync_copy(zeros_hbm.at[pl.ds(s * RPT, RPT)],
                    agg_sh.at[pl.ds(s * RPT, RPT)])
    pltpu.sync_copy(src_hbm.at[w], src_vm)
    pltpu.sync_copy(dst_hbm.at[w], dst_vm)
    plsc.subcore_barrier()

    pltpu.async_copy(p_hbm.at[src_vm.at[0]], buf_a, sem_a)
    pltpu.async_copy(p_hbm.at[src_vm.at[1]], buf_b, sem_b)

    @pl.loop(0, CH, step=2)
    def _chunks(j):
        pltpu.make_async_copy(p_hbm.at[src_vm.at[j]], buf_a, sem_a).wait()
        pltpu.sync_copy(buf_a, agg_sh.at[dst_vm.at[j]], add=True)

        @pl.when(j + 2 < CH)
        def _():
            pltpu.async_copy(p_hbm.at[src_vm.at[j + 2]], buf_a, sem_a)

        pltpu.make_async_copy(p_hbm.at[src_vm.at[j + 1]], buf_b, sem_b).wait()
        pltpu.sync_copy(buf_b, agg_sh.at[dst_vm.at[j + 1]], add=True)

        @pl.when(j + 3 < CH)
        def _():
            pltpu.async_copy(p_hbm.at[src_vm.at[j + 3]], buf_b, sem_b)

    plsc.subcore_barrier()
    pltpu.sync_copy(agg_sh.at[pl.ds(s * RPT, RPT)],
                    out_hbm.at[c, pl.ds(s * RPT, RPT)])


_mlp_call = pl.pallas_call(
    _mlp_body,
    out_shape=jax.ShapeDtypeStruct((NP, OUT_C), jnp.float32),
)

_prep_call = pl.pallas_call(
    _prep_body,
    out_shape=(
        jax.ShapeDtypeStruct((NP, 1), jnp.float32),
        jax.ShapeDtypeStruct((NP, 1), jnp.float32),
        jax.ShapeDtypeStruct((NP, OUT_C), jnp.float32),
    ),
)

_blend_call = pl.pallas_call(
    _blend_body,
    out_shape=(
        jax.ShapeDtypeStruct((NP, OUT_C), jnp.float32),
        jax.ShapeDtypeStruct((NP, OUT_C), jnp.float32),
    ),
)

_lsm_call = pl.pallas_call(
    _lsm_body,
    out_shape=jax.ShapeDtypeStruct((NP, OUT_C), jnp.float32),
)

_deg_call = pl.kernel(
    _deg_body,
    out_type=jax.ShapeDtypeStruct((NC, NP, 16), jnp.float32),
    mesh=_mesh,
    compiler_params=pltpu.CompilerParams(use_tc_tiling_on_sc=False),
    scratch_types=[
        pltpu.VMEM((CH, C), jnp.int32),
        pltpu.VMEM((C, 16), jnp.float32),
        pltpu.VMEM_SHARED((NP, 16), jnp.float32),
    ],
)

_spmm_call = pl.kernel(
    _spmm_body,
    out_type=jax.ShapeDtypeStruct((NC, NP, OUT_C), jnp.float32),
    mesh=_mesh,
    compiler_params=pltpu.CompilerParams(use_tc_tiling_on_sc=False),
    scratch_types=[
        pltpu.VMEM((CH, C), jnp.int32),
        pltpu.VMEM((CH, C), jnp.int32),
        pltpu.VMEM((C, OUT_C), jnp.float32),
        pltpu.VMEM((C, OUT_C), jnp.float32),
        pltpu.VMEM_SHARED((NP, OUT_C), jnp.float32),
        pltpu.SemaphoreType.DMA,
        pltpu.SemaphoreType.DMA,
    ],
)


def kernel(x, edge_index, W1, b1, W2, b2):
    f32 = jnp.float32
    x_pad = jnp.concatenate([x, jnp.zeros((NP - N, IN_C), f32)], axis=0)
    pad = jnp.full((E_PAD - E,), N, jnp.int32)
    srcp = jnp.concatenate([edge_index[0], pad]).reshape(W, CH, C)
    dstp = jnp.concatenate([edge_index[1], pad]).reshape(W, CH, C)

    ones16 = jnp.ones((C, 16), f32)
    zeros16 = jnp.zeros((NP, 16), f32)
    zeros64 = jnp.zeros((NP, OUT_C), f32)

    h = _mlp_call(x_pad, W1, b1.reshape(1, HID), W2, b2.reshape(1, OUT_C))
    deg16 = _deg_call(dstp, ones16, zeros16)
    dinv, dinv2, p = _prep_call(deg16, h)

    out = h
    for _ in range(K):
        agg2 = _spmm_call(p, srcp, dstp, zeros64)
        out, p = _blend_call(agg2, out, h, dinv, dinv2)

    y = _lsm_call(out)
    return y[:N]
